# baseline (speedup 1.0000x reference)
import functools
import math

import jax
import jax.numpy as jnp
from jax import lax
from jax.experimental import pallas as pl
from jax.experimental.pallas import tpu as pltpu
from jax.experimental.pallas import tpu_sc as plsc

_VOCAB = 1000000
_EMBED = 32
_B = 16384
_L = 50
_N = _B * _L
_NC = 2
_NS = 16
_NW = _NC * _NS
_PER_W = _N // _NW
_C = 128
_NCHUNK = _PER_W // _C
_SCALE = math.sqrt(_EMBED)

_mesh = plsc.VectorSubcoreMesh(core_axis_name="c", subcore_axis_name="s")


@functools.partial(
    pl.kernel,
    mesh=_mesh,
    out_type=jax.ShapeDtypeStruct((_N, _EMBED), jnp.float32),
    scratch_types=[
        pltpu.VMEM((_C,), jnp.int32),
        pltpu.VMEM((_C, _EMBED), jnp.float32),
        pltpu.SemaphoreType.DMA,
    ],
    compiler_params=pltpu.CompilerParams(use_tc_tiling_on_sc=False),
)
def _embed_gather(x_hbm, table_hbm, out_hbm, idx_v, rows_v, sem):
    wid = lax.axis_index("s") * _NC + lax.axis_index("c")
    base = wid * _PER_W

    def chunk(j, carry):
        off = base + j * _C
        pltpu.sync_copy(x_hbm.at[pl.ds(off, _C)], idx_v)
        pltpu.async_copy(table_hbm.at[idx_v], rows_v, sem).wait()

        def row(i, c2):
            rows_v[i, pl.ds(0, 16)] = rows_v[i, pl.ds(0, 16)] * _SCALE
            rows_v[i, pl.ds(16, 16)] = rows_v[i, pl.ds(16, 16)] * _SCALE
            return c2

        lax.fori_loop(0, _C, row, 0)
        pltpu.sync_copy(rows_v, out_hbm.at[pl.ds(off, _C)])
        return carry

    lax.fori_loop(0, _NCHUNK, chunk, 0)


def kernel(x, table):
    x_flat = x.reshape(_N).astype(jnp.int32)
    out = _embed_gather(x_flat, table)
    return out.reshape(_B, _L, _EMBED)


# speedup vs baseline: 1.2000x; 1.2000x over previous
import functools
import math

import jax
import jax.numpy as jnp
from jax import lax
from jax.experimental import pallas as pl
from jax.experimental.pallas import tpu as pltpu
from jax.experimental.pallas import tpu_sc as plsc

_VOCAB = 1000000
_EMBED = 32
_B = 16384
_L = 50
_N = _B * _L
_NC = 2
_NS = 16
_NW = _NC * _NS
_PER_W = _N // _NW
_G = 128
_NGB = 4
_K = _G * _NGB
_NBLK = _PER_W // _K
_NBUF = 3
_AHEAD = _NBUF - 1
_SCALE = math.sqrt(_EMBED)

_mesh = plsc.VectorSubcoreMesh(core_axis_name="c", subcore_axis_name="s")


@functools.partial(
    pl.kernel,
    mesh=_mesh,
    out_type=jax.ShapeDtypeStruct((_N, _EMBED), jnp.float32),
    scratch_types=[
        pltpu.VMEM((_NBUF, _K), jnp.int32),
        pltpu.VMEM((_NBUF, _K, _EMBED), jnp.float32),
        pltpu.SemaphoreType.DMA((_NBUF,)),
        pltpu.SemaphoreType.DMA((_NBUF,)),
    ],
    compiler_params=pltpu.CompilerParams(use_tc_tiling_on_sc=False),
)
def _embed_gather(x_hbm, table_hbm, out_hbm, idx_v, rows_v, gsem, osem):
    wid = lax.axis_index("s") * _NC + lax.axis_index("c")
    base = wid * _PER_W

    def issue_block(j, slot):
        off = base + j * _K
        pltpu.sync_copy(x_hbm.at[pl.ds(off, _K)], idx_v.at[slot])
        for k in range(_NGB):
            pltpu.async_copy(
                table_hbm.at[idx_v.at[slot, pl.ds(k * _G, _G)]],
                rows_v.at[slot, pl.ds(k * _G, _G)],
                gsem.at[slot],
            )

    for j in range(_AHEAD):
        issue_block(j, j)

    def step(g, carry):
        slot = lax.rem(g, _NBUF)
        pltpu.make_async_copy(
            table_hbm.at[pl.ds(0, _K)], rows_v.at[slot], gsem.at[slot]
        ).wait()

        rv = rows_v.at[slot]

        @plsc.parallel_loop(0, _K, 1, unroll=8)
        def _scale(i):
            rv[i, pl.ds(0, 16)] = rv[i, pl.ds(0, 16)] * _SCALE
            rv[i, pl.ds(16, 16)] = rv[i, pl.ds(16, 16)] * _SCALE

        off = base + g * _K
        pltpu.async_copy(rows_v.at[slot], out_hbm.at[pl.ds(off, _K)],
                         osem.at[slot])

        j = g + _AHEAD

        @pl.when(j < _NBLK)
        def _prefetch():
            s2 = lax.rem(j, _NBUF)

            @pl.when(g >= 1)
            def _wait_prev_scatter():
                pltpu.make_async_copy(
                    rows_v.at[s2], out_hbm.at[pl.ds(0, _K)], osem.at[s2]
                ).wait()

            off2 = base + j * _K
            pltpu.sync_copy(x_hbm.at[pl.ds(off2, _K)], idx_v.at[s2])
            for k in range(_NGB):
                pltpu.async_copy(
                    table_hbm.at[idx_v.at[s2, pl.ds(k * _G, _G)]],
                    rows_v.at[s2, pl.ds(k * _G, _G)],
                    gsem.at[s2],
                )

        return carry

    lax.fori_loop(0, _NBLK, step, 0)

    for s in range(_NBUF):
        pltpu.make_async_copy(
            rows_v.at[s], out_hbm.at[pl.ds(0, _K)], osem.at[s]
        ).wait()


def kernel(x, table):
    x_flat = x.reshape(_N).astype(jnp.int32)
    out = _embed_gather(x_flat, table)
    return out.reshape(_B, _L, _EMBED)


# speedup vs baseline: 1.9480x; 1.6234x over previous
import functools
import math

import jax
import jax.numpy as jnp
from jax import lax
from jax.experimental import pallas as pl
from jax.experimental.pallas import tpu as pltpu
from jax.experimental.pallas import tpu_sc as plsc

_VOCAB = 1000000
_EMBED = 32
_B = 16384
_L = 50
_NC = 2
_NS = 16
_NW = _NC * _NS
_RW = _B // _NW
_R = 16
_K = _R * _L
_NBLK = _RW // _R
_NBUF = 3
_AHEAD = _NBUF - 1
_SCALE = math.sqrt(_EMBED)

_mesh = plsc.VectorSubcoreMesh(core_axis_name="c", subcore_axis_name="s")


@functools.partial(
    pl.kernel,
    mesh=_mesh,
    out_type=jax.ShapeDtypeStruct((_B, _L, _EMBED), jnp.float32),
    scratch_types=[
        pltpu.VMEM((_NBUF, _R, _L), jnp.int32),
        pltpu.VMEM((_NBUF, _R, _L, _EMBED), jnp.float32),
        pltpu.SemaphoreType.DMA((_NBUF,)),
        pltpu.SemaphoreType.DMA((_NBUF,)),
    ],
    compiler_params=pltpu.CompilerParams(use_tc_tiling_on_sc=False),
)
def _embed_gather(x_hbm, table_hbm, out_hbm, idx_v, rows_v, gsem, osem):
    wid = lax.axis_index("s") * _NC + lax.axis_index("c")
    row0 = wid * _RW

    def issue_block(j, slot):
        r0 = row0 + j * _R
        pltpu.sync_copy(x_hbm.at[pl.ds(r0, _R)], idx_v.at[slot])
        for r in range(_R):
            pltpu.async_copy(
                table_hbm.at[idx_v.at[slot, r]],
                rows_v.at[slot, r],
                gsem.at[slot],
            )

    for j in range(_AHEAD):
        issue_block(j, j)

    def step(g, carry):
        slot = lax.rem(g, _NBUF)
        pltpu.make_async_copy(
            out_hbm.at[pl.ds(0, _R)], rows_v.at[slot], gsem.at[slot]
        ).wait()

        def scale_row(r, c2):
            @plsc.parallel_loop(0, _L, 1, unroll=5)
            def _scale(i):
                rows_v[slot, r, i, pl.ds(0, 16)] = (
                    rows_v[slot, r, i, pl.ds(0, 16)] * _SCALE)
                rows_v[slot, r, i, pl.ds(16, 16)] = (
                    rows_v[slot, r, i, pl.ds(16, 16)] * _SCALE)

            return c2

        lax.fori_loop(0, _R, scale_row, 0)

        r0 = row0 + g * _R
        pltpu.async_copy(rows_v.at[slot], out_hbm.at[pl.ds(r0, _R)],
                         osem.at[slot])

        j = g + _AHEAD

        @pl.when(j < _NBLK)
        def _prefetch():
            s2 = lax.rem(j, _NBUF)

            @pl.when(g >= 1)
            def _wait_prev_scatter():
                pltpu.make_async_copy(
                    rows_v.at[s2], out_hbm.at[pl.ds(0, _R)], osem.at[s2]
                ).wait()

            issue_block(j, s2)

        return carry

    lax.fori_loop(0, _NBLK, step, 0)

    for s in range(_NBUF):
        pltpu.make_async_copy(
            rows_v.at[s], out_hbm.at[pl.ds(0, _R)], osem.at[s]
        ).wait()


def kernel(x, table):
    return _embed_gather(x.astype(jnp.int32), table)


# speedup vs baseline: 1.9492x; 1.0006x over previous
import functools
import math

import jax
import jax.numpy as jnp
from jax import lax
from jax.experimental import pallas as pl
from jax.experimental.pallas import tpu as pltpu
from jax.experimental.pallas import tpu_sc as plsc

_VOCAB = 1000000
_EMBED = 32
_B = 16384
_L = 50
_NC = 2
_NS = 16
_NW = _NC * _NS
_RW = _B // _NW
_R = 16
_K = _R * _L
_NBLK = _RW // _R
_NBUF = 3
_AHEAD = _NBUF - 1
_SCALE = math.sqrt(_EMBED)

_mesh = plsc.VectorSubcoreMesh(core_axis_name="c", subcore_axis_name="s")


@functools.partial(
    pl.kernel,
    mesh=_mesh,
    out_type=jax.ShapeDtypeStruct((_B, _L, _EMBED), jnp.float32),
    scratch_types=[
        pltpu.VMEM((_NBUF, _R, _L), jnp.int32),
        pltpu.VMEM((_NBUF, _R, _L, _EMBED), jnp.float32),
        pltpu.SemaphoreType.DMA((_NBUF,)),
        pltpu.SemaphoreType.DMA((_NBUF,)),
    ],
    compiler_params=pltpu.CompilerParams(
        use_tc_tiling_on_sc=False, needs_layout_passes=True),
)
def _embed_gather(x_hbm, table_hbm, out_hbm, idx_v, rows_v, gsem, osem):
    wid = lax.axis_index("s") * _NC + lax.axis_index("c")
    row0 = wid * _RW

    def issue_block(j, slot):
        r0 = row0 + j * _R
        pltpu.sync_copy(x_hbm.at[pl.ds(r0, _R)], idx_v.at[slot])
        for r in range(_R):
            pltpu.async_copy(
                table_hbm.at[idx_v.at[slot, r]],
                rows_v.at[slot, r],
                gsem.at[slot],
            )

    for j in range(_AHEAD):
        issue_block(j, j)

    def step(g, carry):
        slot = lax.rem(g, _NBUF)
        pltpu.make_async_copy(
            out_hbm.at[pl.ds(0, _R)], rows_v.at[slot], gsem.at[slot]
        ).wait()

        def scale_row(r, c2):
            @plsc.parallel_loop(0, _L, 1, unroll=5)
            def _scale(i):
                rows_v[slot, r, i, pl.ds(0, 16)] = (
                    rows_v[slot, r, i, pl.ds(0, 16)] * _SCALE)
                rows_v[slot, r, i, pl.ds(16, 16)] = (
                    rows_v[slot, r, i, pl.ds(16, 16)] * _SCALE)

            return c2

        lax.fori_loop(0, _R, scale_row, 0)

        r0 = row0 + g * _R
        pltpu.async_copy(rows_v.at[slot], out_hbm.at[pl.ds(r0, _R)],
                         osem.at[slot])

        j = g + _AHEAD

        @pl.when(j < _NBLK)
        def _prefetch():
            s2 = lax.rem(j, _NBUF)

            @pl.when(g >= 1)
            def _wait_prev_scatter():
                pltpu.make_async_copy(
                    rows_v.at[s2], out_hbm.at[pl.ds(0, _R)], osem.at[s2]
                ).wait()

            issue_block(j, s2)

        return carry

    lax.fori_loop(0, _NBLK, step, 0)

    for s in range(_NBUF):
        pltpu.make_async_copy(
            rows_v.at[s], out_hbm.at[pl.ds(0, _R)], osem.at[s]
        ).wait()


def kernel(x, table):
    return _embed_gather(x.astype(jnp.int32), table)


# speedup vs baseline: 2.0221x; 1.0374x over previous
import functools
import math

import jax
import jax.numpy as jnp
from jax import lax
from jax.experimental import pallas as pl
from jax.experimental.pallas import tpu as pltpu
from jax.experimental.pallas import tpu_sc as plsc

_VOCAB = 1000000
_EMBED = 32
_B = 16384
_L = 50
_NC = 2
_NS = 16
_NW = _NC * _NS
_S = 4
_BC = _B // _S
_R = 16
_NBUF = 3
_AHEAD = _NBUF - 1
_SCALE = math.sqrt(_EMBED)

_mesh = plsc.VectorSubcoreMesh(core_axis_name="c", subcore_axis_name="s")


def _make_chunk(nrows):
    rw = nrows // _NW
    nblk = rw // _R

    @functools.partial(
        pl.kernel,
        mesh=_mesh,
        out_type=jax.ShapeDtypeStruct((nrows, _L, _EMBED), jnp.float32),
        scratch_types=[
            pltpu.VMEM((_NBUF, _R, _L), jnp.int32),
            pltpu.VMEM((_NBUF, _R, _L, _EMBED), jnp.float32),
            pltpu.SemaphoreType.DMA((_NBUF,)),
            pltpu.SemaphoreType.DMA((_NBUF,)),
        ],
        compiler_params=pltpu.CompilerParams(use_tc_tiling_on_sc=False),
    )
    def _embed_gather(x_hbm, table_hbm, out_hbm, idx_v, rows_v, gsem, osem):
        wid = lax.axis_index("s") * _NC + lax.axis_index("c")
        row0 = wid * rw

        def issue_block(j, slot):
            r0 = row0 + j * _R
            pltpu.sync_copy(x_hbm.at[pl.ds(r0, _R)], idx_v.at[slot])
            for r in range(_R):
                pltpu.async_copy(
                    table_hbm.at[idx_v.at[slot, r]],
                    rows_v.at[slot, r],
                    gsem.at[slot],
                )

        for j in range(_AHEAD):
            issue_block(j, j)

        def step(g, carry):
            slot = lax.rem(g, _NBUF)
            pltpu.make_async_copy(
                out_hbm.at[pl.ds(0, _R)], rows_v.at[slot], gsem.at[slot]
            ).wait()

            def scale_row(r, c2):
                @plsc.parallel_loop(0, _L, 1, unroll=5)
                def _scale(i):
                    rows_v[slot, r, i, pl.ds(0, 16)] = (
                        rows_v[slot, r, i, pl.ds(0, 16)] * _SCALE)
                    rows_v[slot, r, i, pl.ds(16, 16)] = (
                        rows_v[slot, r, i, pl.ds(16, 16)] * _SCALE)

                return c2

            lax.fori_loop(0, _R, scale_row, 0)

            r0 = row0 + g * _R
            pltpu.async_copy(rows_v.at[slot], out_hbm.at[pl.ds(r0, _R)],
                             osem.at[slot])

            j = g + _AHEAD

            @pl.when(j < nblk)
            def _prefetch():
                s2 = lax.rem(j, _NBUF)

                @pl.when(g >= 1)
                def _wait_prev_scatter():
                    pltpu.make_async_copy(
                        rows_v.at[s2], out_hbm.at[pl.ds(0, _R)], osem.at[s2]
                    ).wait()

                issue_block(j, s2)

            return carry

        lax.fori_loop(0, nblk, step, 0)

        for s in range(_NBUF):
            pltpu.make_async_copy(
                rows_v.at[s], out_hbm.at[pl.ds(0, _R)], osem.at[s]
            ).wait()

    return _embed_gather


_chunk_kernel = _make_chunk(_BC)


def kernel(x, table):
    xi = x.astype(jnp.int32)
    outs = [_chunk_kernel(xi[c * _BC:(c + 1) * _BC], table)
            for c in range(_S)]
    return jnp.concatenate(outs, axis=0)
